# Initial kernel scaffold; baseline (speedup 1.0000x reference)
import jax
import jax.numpy as jnp
from jax.experimental import pallas as pl


def kernel(x, table):
    raise NotImplementedError("write your pallas kernel here")


# speedup vs baseline: 3.8028x; 3.8028x over previous
import functools

import jax
import jax.numpy as jnp
from jax import lax
from jax.experimental import pallas as pl
from jax.experimental.pallas import tpu as pltpu
from jax.experimental.pallas import tpu_sc as plsc

NC, NS, L = 2, 16, 16
NW = NC * NS
B, SEQ, D = 1024, 200, 128
BPW = B // NW
CH = 40
CPS = SEQ // CH
NCHUNK = BPW * CPS
ND = D // L

_mesh = plsc.VectorSubcoreMesh(
    core_axis_name="c", subcore_axis_name="s", num_cores=NC, num_subcores=NS
)


@functools.partial(
    pl.kernel,
    out_type=jax.ShapeDtypeStruct((B, D), jnp.float32),
    mesh=_mesh,
    scratch_types=[
        pltpu.VMEM((NCHUNK, CH), jnp.int32),
        pltpu.VMEM((CH, D), jnp.float32),
        pltpu.VMEM((BPW, D), jnp.float32),
        pltpu.SemaphoreType.DMA,
    ],
)
def _encode(x_hbm, table_hbm, out_hbm, idx_v, rows_v, out_v, sem):
    wid = lax.axis_index("s") * NC + lax.axis_index("c")

    pltpu.sync_copy(x_hbm.at[pl.ds(wid * NCHUNK, NCHUNK)], idx_v)

    def zero_body(i, carry):
        for j in range(ND):
            out_v[i, pl.ds(j * L, L)] = jnp.zeros((L,), jnp.float32)
        return carry

    lax.fori_loop(0, BPW, zero_body, 0)

    def chunk_body(g, carry):
        s = g // CPS
        pltpu.async_copy(table_hbm.at[idx_v.at[g]], rows_v, sem).wait()

        def row_body(r, acc):
            return tuple(acc[j] + rows_v[r, pl.ds(j * L, L)] for j in range(ND))

        acc = lax.fori_loop(
            0, CH, row_body,
            tuple(jnp.zeros((L,), jnp.float32) for _ in range(ND)),
        )
        for j in range(ND):
            plsc.addupdate(out_v.at[s, pl.ds(j * L, L)], acc[j])
        return carry

    lax.fori_loop(0, NCHUNK, chunk_body, 0)
    pltpu.sync_copy(out_v, out_hbm.at[pl.ds(wid * BPW, BPW)])


def kernel(x, table):
    x2 = x.reshape(B * CPS, CH).astype(jnp.int32)
    return _encode(x2, table)


# speedup vs baseline: 6.5445x; 1.7210x over previous
import functools

import jax
import jax.numpy as jnp
from jax import lax
from jax.experimental import pallas as pl
from jax.experimental.pallas import tpu as pltpu
from jax.experimental.pallas import tpu_sc as plsc

NC, NS, L = 2, 16, 16
NW = NC * NS
B, SEQ, D = 1024, 200, 128
BPW = B // NW
CH = 40
CPS = SEQ // CH
NCHUNK = BPW * CPS
ND = D // L

_mesh = plsc.VectorSubcoreMesh(
    core_axis_name="c", subcore_axis_name="s", num_cores=NC, num_subcores=NS
)


@functools.partial(
    pl.kernel,
    out_type=jax.ShapeDtypeStruct((B, D), jnp.float32),
    mesh=_mesh,
    scratch_types=[
        pltpu.VMEM((NCHUNK, CH), jnp.int32),
        pltpu.VMEM((CH, D), jnp.float32),
        pltpu.VMEM((CH, D), jnp.float32),
        pltpu.VMEM((BPW, D), jnp.float32),
        pltpu.SemaphoreType.DMA,
        pltpu.SemaphoreType.DMA,
    ],
)
def _encode(x_hbm, table_hbm, out_hbm, idx_v, rows0, rows1, out_v, sem0, sem1):
    wid = lax.axis_index("s") * NC + lax.axis_index("c")
    rows = (rows0, rows1)
    sems = (sem0, sem1)

    pltpu.sync_copy(x_hbm.at[pl.ds(wid * NCHUNK, NCHUNK)], idx_v)

    def zero_body(i, carry):
        for j in range(ND):
            out_v[i, pl.ds(j * L, L)] = jnp.zeros((L,), jnp.float32)
        return carry

    lax.fori_loop(0, BPW, zero_body, 0)

    def reduce_chunk(g, buf):
        s = g // CPS

        def row_body(r, acc):
            return tuple(acc[j] + buf[r, pl.ds(j * L, L)] for j in range(ND))

        acc = lax.fori_loop(
            0, CH, row_body,
            tuple(jnp.zeros((L,), jnp.float32) for _ in range(ND)),
        )
        for j in range(ND):
            plsc.addupdate(out_v.at[s, pl.ds(j * L, L)], acc[j])

    pltpu.async_copy(table_hbm.at[idx_v.at[0]], rows[0], sems[0])

    def pair_body(i, carry):
        g = 2 * i
        d1 = pltpu.async_copy(table_hbm.at[idx_v.at[g + 1]], rows[1], sems[1])
        pltpu.make_async_copy(table_hbm.at[idx_v.at[g]], rows[0], sems[0]).wait()
        reduce_chunk(g, rows[0])

        @pl.when(i + 1 < NCHUNK // 2)
        def _():
            pltpu.async_copy(table_hbm.at[idx_v.at[g + 2]], rows[0], sems[0])

        d1.wait()
        reduce_chunk(g + 1, rows[1])
        return carry

    lax.fori_loop(0, NCHUNK // 2, pair_body, 0)
    pltpu.sync_copy(out_v, out_hbm.at[pl.ds(wid * BPW, BPW)])


def kernel(x, table):
    x2 = x.reshape(B * CPS, CH).astype(jnp.int32)
    return _encode(x2, table)
